# baseline (speedup 1.0000x reference)
import functools

import jax
import jax.numpy as jnp
from jax import lax
from jax.experimental import pallas as pl
from jax.experimental.pallas import tpu as pltpu
from jax.experimental.pallas import tpu_sc as plsc

NUM_CORES = 2
NUM_SUBCORES = 16
NUM_WORKERS = NUM_CORES * NUM_SUBCORES
IDX_W = 128
RING = 8
DEPTH = 6
CPB = 32
PAD_W = 72


@functools.lru_cache(maxsize=None)
def _build_gather(vocab: int, dim: int, batch: int, seq: int):
  assert batch % (NUM_WORKERS * RING) == 0
  assert seq % 8 == 0 and seq > IDX_W and seq <= 2 * IDX_W
  seq_hi = seq - IDX_W
  rows_per_w = batch // NUM_WORKERS
  n_chunks = rows_per_w
  n_groups = n_chunks // RING
  n_blocks = n_chunks // CPB
  assert n_chunks % CPB == 0

  mesh = plsc.VectorSubcoreMesh(
      core_axis_name="c", subcore_axis_name="s",
      num_cores=NUM_CORES, num_subcores=NUM_SUBCORES)

  @functools.partial(
      pl.kernel,
      out_type=jax.ShapeDtypeStruct((batch, seq, PAD_W), jnp.float32),
      mesh=mesh,
      compiler_params=pltpu.CompilerParams(use_tc_tiling_on_sc=False),
      scratch_types=[
          pltpu.VMEM((2, CPB, seq), jnp.int32),
          pltpu.VMEM((RING, seq, dim), jnp.float32),
          pltpu.SemaphoreType.DMA,
          pltpu.SemaphoreType.DMA,
          pltpu.SemaphoreType.DMA,
          pltpu.SemaphoreType.DMA,
          pltpu.SemaphoreType.DMA,
          pltpu.SemaphoreType.DMA,
          pltpu.SemaphoreType.DMA,
          pltpu.SemaphoreType.DMA,
          pltpu.SemaphoreType.DMA,
          pltpu.SemaphoreType.DMA,
          pltpu.SemaphoreType.DMA,
          pltpu.SemaphoreType.DMA,
          pltpu.SemaphoreType.DMA,
          pltpu.SemaphoreType.DMA,
          pltpu.SemaphoreType.DMA,
          pltpu.SemaphoreType.DMA,
          pltpu.SemaphoreType.DMA,
      ],
  )
  def gather_kernel(table_hbm, idx_hbm, out_hbm, idx_v, rows_v, sem_i,
                    sg0, sg1, sg2, sg3, sg4, sg5, sg6, sg7,
                    sw0, sw1, sw2, sw3, sw4, sw5, sw6, sw7):
    wid = lax.axis_index("s") * NUM_CORES + lax.axis_index("c")
    row0 = wid * rows_per_w
    sg = (sg0, sg1, sg2, sg3, sg4, sg5, sg6, sg7)
    sw = (sw0, sw1, sw2, sw3, sw4, sw5, sw6, sw7)

    def idx_copy(blk):
      return pltpu.make_async_copy(
          idx_hbm.at[pl.ds(row0 + blk * CPB, CPB)],
          idx_v.at[blk % 2], sem_i)

    def gather_copies(c, b):
      slot, r = (c // CPB) % 2, c % CPB
      return (
          pltpu.make_async_copy(
              table_hbm.at[idx_v.at[slot, r, pl.ds(0, IDX_W)]],
              rows_v.at[b, pl.ds(0, IDX_W)], sg[b]),
          pltpu.make_async_copy(
              table_hbm.at[idx_v.at[slot, r, pl.ds(IDX_W, seq_hi)]],
              rows_v.at[b, pl.ds(IDX_W, seq_hi)], sg[b]),
      )

    def wb_copy(c, b):
      return pltpu.make_async_copy(
          rows_v.at[b],
          out_hbm.at[row0 + c, pl.ds(0, seq), pl.ds(0, dim)], sw[b])

    def start_gather(c, b):
      @pl.when(c % CPB == 0)
      def _():
        blk = c // CPB
        idx_copy(blk).wait()

        @pl.when(blk + 1 < n_blocks)
        def _():
          idx_copy(blk + 1).start()

      for cp in gather_copies(c, b):
        cp.start()

    idx_copy(0).start()
    for b in range(DEPTH):
      start_gather(b, b)

    @pl.loop(0, n_groups)
    def _group(q):
      c0 = q * RING
      for b in range(RING):
        c = c0 + b
        for cp in gather_copies(c, b):
          cp.wait()
        wb_copy(c, b).start()
        f = c + DEPTH
        fb = (b + DEPTH) % RING

        @pl.when(f < n_chunks)
        def _():
          @pl.when(f >= RING)
          def _():
            wb_copy(f - RING, fb).wait()
          start_gather(f, fb)

    for b in range(RING):
      wb_copy(n_chunks - RING + b, b).wait()

  return gather_kernel


def kernel(smiles_indices, embedding_table):
  batch, seq = smiles_indices.shape
  vocab, dim = embedding_table.shape
  gather = _build_gather(vocab, dim, batch, seq)
  padded = gather(embedding_table, smiles_indices.astype(jnp.int32))
  return padded[:, :, :dim]


# speedup vs baseline: 1.9175x; 1.9175x over previous
import functools

import jax
import jax.numpy as jnp
from jax import lax
from jax.experimental import pallas as pl
from jax.experimental.pallas import tpu as pltpu
from jax.experimental.pallas import tpu_sc as plsc

NUM_CORES = 2
NUM_SUBCORES = 16
NUM_WORKERS = NUM_CORES * NUM_SUBCORES
IDX_W = 128
RING = 8
DEPTH = 6
CPB = 32
PAD_W = 128


@functools.lru_cache(maxsize=None)
def _build_gather(vocab: int, dim: int, batch: int, seq: int):
  assert batch % (NUM_WORKERS * RING) == 0
  assert seq % 8 == 0 and seq > IDX_W and seq <= 2 * IDX_W
  seq_hi = seq - IDX_W
  rows_per_w = batch // NUM_WORKERS
  n_chunks = rows_per_w
  n_groups = n_chunks // RING
  n_blocks = n_chunks // CPB
  assert n_chunks % CPB == 0

  mesh = plsc.VectorSubcoreMesh(
      core_axis_name="c", subcore_axis_name="s",
      num_cores=NUM_CORES, num_subcores=NUM_SUBCORES)

  @functools.partial(
      pl.kernel,
      out_type=jax.ShapeDtypeStruct((batch, seq, PAD_W), jnp.float32),
      mesh=mesh,
      compiler_params=pltpu.CompilerParams(use_tc_tiling_on_sc=False),
      scratch_types=[
          pltpu.VMEM((2, CPB, seq), jnp.int32),
          pltpu.VMEM((RING, seq, dim), jnp.float32),
          pltpu.SemaphoreType.DMA,
          pltpu.SemaphoreType.DMA,
          pltpu.SemaphoreType.DMA,
          pltpu.SemaphoreType.DMA,
          pltpu.SemaphoreType.DMA,
          pltpu.SemaphoreType.DMA,
          pltpu.SemaphoreType.DMA,
          pltpu.SemaphoreType.DMA,
          pltpu.SemaphoreType.DMA,
          pltpu.SemaphoreType.DMA,
          pltpu.SemaphoreType.DMA,
          pltpu.SemaphoreType.DMA,
          pltpu.SemaphoreType.DMA,
          pltpu.SemaphoreType.DMA,
          pltpu.SemaphoreType.DMA,
          pltpu.SemaphoreType.DMA,
          pltpu.SemaphoreType.DMA,
      ],
  )
  def gather_kernel(table_hbm, idx_hbm, out_hbm, idx_v, rows_v, sem_i,
                    sg0, sg1, sg2, sg3, sg4, sg5, sg6, sg7,
                    sw0, sw1, sw2, sw3, sw4, sw5, sw6, sw7):
    wid = lax.axis_index("s") * NUM_CORES + lax.axis_index("c")
    row0 = wid * rows_per_w
    sg = (sg0, sg1, sg2, sg3, sg4, sg5, sg6, sg7)
    sw = (sw0, sw1, sw2, sw3, sw4, sw5, sw6, sw7)

    def idx_copy(blk):
      return pltpu.make_async_copy(
          idx_hbm.at[pl.ds(row0 + blk * CPB, CPB)],
          idx_v.at[blk % 2], sem_i)

    def gather_copies(c, b):
      slot, r = (c // CPB) % 2, c % CPB
      return (
          pltpu.make_async_copy(
              table_hbm.at[idx_v.at[slot, r, pl.ds(0, IDX_W)]],
              rows_v.at[b, pl.ds(0, IDX_W)], sg[b]),
          pltpu.make_async_copy(
              table_hbm.at[idx_v.at[slot, r, pl.ds(IDX_W, seq_hi)]],
              rows_v.at[b, pl.ds(IDX_W, seq_hi)], sg[b]),
      )

    def wb_copy(c, b):
      return pltpu.make_async_copy(
          rows_v.at[b],
          out_hbm.at[row0 + c, pl.ds(0, seq), pl.ds(0, dim)], sw[b])

    def start_gather(c, b):
      @pl.when(c % CPB == 0)
      def _():
        blk = c // CPB
        idx_copy(blk).wait()

        @pl.when(blk + 1 < n_blocks)
        def _():
          idx_copy(blk + 1).start()

      for cp in gather_copies(c, b):
        cp.start()

    idx_copy(0).start()
    for b in range(DEPTH):
      start_gather(b, b)

    @pl.loop(0, n_groups)
    def _group(q):
      c0 = q * RING
      for b in range(RING):
        c = c0 + b
        for cp in gather_copies(c, b):
          cp.wait()
        wb_copy(c, b).start()
        f = c + DEPTH
        fb = (b + DEPTH) % RING

        @pl.when(f < n_chunks)
        def _():
          @pl.when(f >= RING)
          def _():
            wb_copy(f - RING, fb).wait()
          start_gather(f, fb)

    for b in range(RING):
      wb_copy(n_chunks - RING + b, b).wait()

  return gather_kernel


def kernel(smiles_indices, embedding_table):
  batch, seq = smiles_indices.shape
  vocab, dim = embedding_table.shape
  gather = _build_gather(vocab, dim, batch, seq)
  padded = gather(embedding_table, smiles_indices.astype(jnp.int32))
  return padded[:, :, :dim]


# speedup vs baseline: 1.9270x; 1.0049x over previous
import functools

import jax
import jax.numpy as jnp
from jax import lax
from jax.experimental import pallas as pl
from jax.experimental.pallas import tpu as pltpu
from jax.experimental.pallas import tpu_sc as plsc

NUM_CORES = 2
NUM_SUBCORES = 16
NUM_WORKERS = NUM_CORES * NUM_SUBCORES
IDX_W = 128
RING = 8
DEPTH = 6
CPB = 32
PAD_W = 128


@functools.lru_cache(maxsize=None)
def _build_gather(vocab: int, dim: int, batch: int, seq: int):
  assert batch % (NUM_WORKERS * RING) == 0
  assert seq % 8 == 0 and seq > IDX_W and seq <= 2 * IDX_W
  seq_hi = seq - IDX_W
  rows_per_w = batch // NUM_WORKERS
  n_chunks = rows_per_w
  n_groups = n_chunks // RING
  n_blocks = n_chunks // CPB
  assert n_chunks % CPB == 0

  mesh = plsc.VectorSubcoreMesh(
      core_axis_name="c", subcore_axis_name="s",
      num_cores=NUM_CORES, num_subcores=NUM_SUBCORES)

  @functools.partial(
      pl.kernel,
      out_type=jax.ShapeDtypeStruct((batch, seq, PAD_W), jnp.float32),
      mesh=mesh,
      compiler_params=pltpu.CompilerParams(use_tc_tiling_on_sc=False),
      scratch_types=[
          pltpu.VMEM((2, CPB, seq), jnp.int32),
          pltpu.VMEM((RING, seq, dim), jnp.float32),
          pltpu.SemaphoreType.DMA,
          pltpu.SemaphoreType.DMA,
          pltpu.SemaphoreType.DMA,
          pltpu.SemaphoreType.DMA,
          pltpu.SemaphoreType.DMA,
          pltpu.SemaphoreType.DMA,
          pltpu.SemaphoreType.DMA,
          pltpu.SemaphoreType.DMA,
          pltpu.SemaphoreType.DMA,
          pltpu.SemaphoreType.DMA,
          pltpu.SemaphoreType.DMA,
          pltpu.SemaphoreType.DMA,
          pltpu.SemaphoreType.DMA,
          pltpu.SemaphoreType.DMA,
          pltpu.SemaphoreType.DMA,
          pltpu.SemaphoreType.DMA,
          pltpu.SemaphoreType.DMA,
      ],
  )
  def gather_kernel(table_hbm, idx_hbm, out_hbm, idx_v, rows_v, sem_i,
                    sg0, sg1, sg2, sg3, sg4, sg5, sg6, sg7,
                    sw0, sw1, sw2, sw3, sw4, sw5, sw6, sw7):
    wid = lax.axis_index("s") * NUM_CORES + lax.axis_index("c")
    row0 = wid * rows_per_w
    sg = (sg0, sg1, sg2, sg3, sg4, sg5, sg6, sg7)
    sw = (sw0, sw1, sw2, sw3, sw4, sw5, sw6, sw7)

    def idx_copy(blk):
      return pltpu.make_async_copy(
          idx_hbm.at[pl.ds(row0 + blk * CPB, CPB)],
          idx_v.at[blk % 2], sem_i)

    def gather_copies(c, b):
      slot, r = (c // CPB) % 2, c % CPB
      return (
          pltpu.make_async_copy(
              table_hbm.at[idx_v.at[slot, r, pl.ds(0, IDX_W)]],
              rows_v.at[b, pl.ds(0, IDX_W)], sg[b]),
          pltpu.make_async_copy(
              table_hbm.at[idx_v.at[slot, r, pl.ds(IDX_W, seq_hi)]],
              rows_v.at[b, pl.ds(IDX_W, seq_hi)], sg[b]),
      )

    def wb_copy(c, b):
      return pltpu.make_async_copy(
          rows_v.at[b],
          out_hbm.at[row0 + c, pl.ds(0, seq), pl.ds(0, dim)], sw[b])

    def start_gather(c, b):
      @pl.when(c % CPB == 0)
      def _():
        blk = c // CPB
        idx_copy(blk).wait()

        @pl.when(blk + 1 < n_blocks)
        def _():
          idx_copy(blk + 1).start()

      for cp in gather_copies(c, b):
        cp.start()

    idx_copy(0).start()
    for b in range(DEPTH):
      start_gather(b, b)

    @pl.loop(0, n_groups)
    def _group(q):
      c0 = q * RING
      for b in range(RING):
        c = c0 + b
        for cp in gather_copies(c, b):
          cp.wait()
        wb_copy(c, b).start()
        f = c + DEPTH
        fb = (b + DEPTH) % RING

        @pl.when(f < n_chunks)
        def _():
          @pl.when(f >= RING)
          def _():
            wb_copy(f - RING, fb).wait()
          start_gather(f, fb)

    for b in range(RING):
      wb_copy(n_chunks - RING + b, b).wait()

  return gather_kernel


def kernel(smiles_indices, embedding_table):
  batch, seq = smiles_indices.shape
  vocab, dim = embedding_table.shape
  gather = _build_gather(vocab, dim, batch, seq)
  padded = gather(embedding_table, smiles_indices.astype(jnp.int32))
  return padded[:, :, :dim] * jnp.float32(1.0)
